# baseline (speedup 1.0000x reference)
import jax
import jax.numpy as jnp
from jax import lax
from jax.experimental import pallas as pl
from jax.experimental.pallas import tpu as pltpu
from jax.experimental.pallas import tpu_sc as plsc

_Z = 1.0
_NEWTON_ITERS = 16
_L = 16
_U = 8
_ROWS_PER_TEC = 4


def _sc_body(x_hbm, wp_hbm, wc_hbm, xbuf, wpbuf, wcbuf, glist, sem_wp, sem_wc):
    n = x_hbm.shape[-1]
    nchunks = n // _L
    ngroups = nchunks // _U
    wid = lax.axis_index("s") * 2 + lax.axis_index("c")

    pending = None
    for r in range(_ROWS_PER_TEC):
        row = wid * _ROWS_PER_TEC + r
        pltpu.sync_copy(x_hbm.at[row], xbuf)

        neg_inf = jnp.full((_L,), -jnp.inf, jnp.float32)

        def pa(g, carry):
            ts = list(carry[:_U])
            gcnt = carry[_U]
            base = g * (_U * _L)
            anym = None
            for u in range(_U):
                v = xbuf[pl.ds(base + u * _L, _L)]
                ts[u] = jnp.maximum(ts[u], v - _Z)
                m = v > ts[u]
                anym = m if anym is None else anym | m
            hit = jnp.any(anym)

            @pl.when(hit)
            def _():
                glist[gcnt] = g

            return tuple(ts) + (gcnt + jnp.where(hit, 1, 0),)

        fin = lax.fori_loop(0, ngroups, pa, (neg_inf,) * _U + (jnp.int32(0),))
        ngr = fin[_U]
        t = fin[0]
        for u in range(1, _U):
            t = jnp.maximum(t, fin[u])
        tau0 = jnp.full((_L,), jnp.max(t), jnp.float32)

        if pending is not None:
            pending[1].wait()

        zero_f = jnp.zeros((_L,), jnp.float32)

        def nstep(tau):
            def inner(i, acc):
                svs = list(acc[:_U])
                kvs = list(acc[_U:])
                base = glist[i] * (_U * _L)
                for u in range(_U):
                    v = xbuf[pl.ds(base + u * _L, _L)]
                    act = v > tau
                    svs[u] = svs[u] + jnp.where(act, v, 0.0)
                    kvs[u] = kvs[u] + jnp.where(act, 1.0, 0.0)
                return tuple(svs) + tuple(kvs)

            acc = lax.fori_loop(0, ngr, inner, (zero_f,) * (2 * _U))
            sv = acc[0]
            kv = acc[_U]
            for u in range(1, _U):
                sv = sv + acc[u]
                kv = kv + acc[_U + u]
            s = jnp.full((_L,), jnp.sum(sv), jnp.float32)
            k = jnp.full((_L,), jnp.sum(kv), jnp.float32)
            return (s - _Z) / k

        def ncond(st):
            i, tau, taup = st
            return (i < _NEWTON_ITERS) & jnp.any(tau != taup)

        def nbody(st):
            i, tau, _ = st
            return (i + 1, nstep(tau), tau)

        _, tau, _ = lax.while_loop(
            ncond, nbody, (jnp.int32(0), nstep(tau0), tau0)
        )

        if pending is not None:
            pending[0].wait()

        def pc(g, _):
            base = g * (_U * _L)
            for u in range(_U):
                sl = pl.ds(base + u * _L, _L)
                v = xbuf[sl]
                wpbuf[sl] = jnp.maximum(v - tau, 0.0)
                wcbuf[sl] = jnp.minimum(v, tau)
            return 0

        lax.fori_loop(0, ngroups, pc, 0)
        cp_wp = pltpu.async_copy(wpbuf, wp_hbm.at[row], sem_wp)
        cp_wc = pltpu.async_copy(wcbuf, wc_hbm.at[row], sem_wc)
        pending = (cp_wp, cp_wc)

    pending[0].wait()
    pending[1].wait()


def kernel(x):
    b, n = x.shape
    mesh = plsc.VectorSubcoreMesh(core_axis_name="c", subcore_axis_name="s")
    out = jax.ShapeDtypeStruct((b, n), jnp.float32)
    f = pl.kernel(
        _sc_body,
        out_type=(out, out),
        mesh=mesh,
        scratch_types=[
            pltpu.VMEM((n,), jnp.float32),
            pltpu.VMEM((n,), jnp.float32),
            pltpu.VMEM((n,), jnp.float32),
            pltpu.SMEM((n // (_U * _L),), jnp.int32),
            pltpu.SemaphoreType.DMA,
            pltpu.SemaphoreType.DMA,
        ],
        compiler_params=pltpu.CompilerParams(needs_layout_passes=False),
    )
    return f(x)


# speedup vs baseline: 1.5302x; 1.5302x over previous
import jax
import jax.numpy as jnp
from jax import lax
from jax.experimental import pallas as pl
from jax.experimental.pallas import tpu as pltpu
from jax.experimental.pallas import tpu_sc as plsc

_Z = 1.0
_NEWTON_ITERS = 16
_L = 16
_U = 8
_ROWS_PER_TEC = 4


def _sc_body(
    x_hbm, wp_hbm, wc_hbm, xbuf, wpbuf, wcbuf, gmax, glist, sem_wp, sem_wc
):
    n = x_hbm.shape[-1]
    nchunks = n // _L
    ngroups = nchunks // _U
    wid = lax.axis_index("s") * 2 + lax.axis_index("c")

    pending = None
    for r in range(_ROWS_PER_TEC):
        row = wid * _ROWS_PER_TEC + r
        pltpu.sync_copy(x_hbm.at[row], xbuf)

        neg_inf = jnp.full((_L,), -jnp.inf, jnp.float32)

        def pa(g, ms):
            ms = list(ms)
            base = g * (_U * _L)
            gm = None
            for u in range(_U):
                v = xbuf[pl.ds(base + u * _L, _L)]
                ms[u] = jnp.maximum(ms[u], v)
                gm = v if gm is None else jnp.maximum(gm, v)
            gmax[pl.ds(g * _L, _L)] = gm
            return tuple(ms)

        ms = lax.fori_loop(0, ngroups, pa, (neg_inf,) * _U)
        m = ms[0]
        for u in range(1, _U):
            m = jnp.maximum(m, ms[u])
        tau0 = jnp.full((_L,), jnp.max(m) - _Z, jnp.float32)

        def pb(g, gcnt):
            hit = jnp.any(gmax[pl.ds(g * _L, _L)] > tau0)

            @pl.when(hit)
            def _():
                glist[gcnt] = g

            return gcnt + jnp.where(hit, 1, 0)

        ngr = lax.fori_loop(0, ngroups, pb, jnp.int32(0))

        if pending is not None:
            pending[1].wait()

        zero_f = jnp.zeros((_L,), jnp.float32)

        def nstep(tau):
            def inner(i, acc):
                svs = list(acc[:_U])
                kvs = list(acc[_U:])
                base = glist[i] * (_U * _L)
                for u in range(_U):
                    v = xbuf[pl.ds(base + u * _L, _L)]
                    act = v > tau
                    svs[u] = svs[u] + jnp.where(act, v, 0.0)
                    kvs[u] = kvs[u] + jnp.where(act, 1.0, 0.0)
                return tuple(svs) + tuple(kvs)

            acc = lax.fori_loop(0, ngr, inner, (zero_f,) * (2 * _U))
            sv = acc[0]
            kv = acc[_U]
            for u in range(1, _U):
                sv = sv + acc[u]
                kv = kv + acc[_U + u]
            s = jnp.full((_L,), jnp.sum(sv), jnp.float32)
            k = jnp.full((_L,), jnp.sum(kv), jnp.float32)
            return (s - _Z) / k

        def ncond(st):
            i, tau, taup = st
            return (i < _NEWTON_ITERS) & jnp.any(tau != taup)

        def nbody(st):
            i, tau, _ = st
            return (i + 1, nstep(tau), tau)

        _, tau, _ = lax.while_loop(
            ncond, nbody, (jnp.int32(0), nstep(tau0), tau0)
        )

        if pending is not None:
            pending[0].wait()

        def pc(g, _):
            base = g * (_U * _L)
            for u in range(_U):
                sl = pl.ds(base + u * _L, _L)
                v = xbuf[sl]
                wpbuf[sl] = jnp.maximum(v - tau, 0.0)
                wcbuf[sl] = jnp.minimum(v, tau)
            return 0

        lax.fori_loop(0, ngroups, pc, 0)
        cp_wp = pltpu.async_copy(wpbuf, wp_hbm.at[row], sem_wp)
        cp_wc = pltpu.async_copy(wcbuf, wc_hbm.at[row], sem_wc)
        pending = (cp_wp, cp_wc)

    pending[0].wait()
    pending[1].wait()


def kernel(x):
    b, n = x.shape
    mesh = plsc.VectorSubcoreMesh(core_axis_name="c", subcore_axis_name="s")
    out = jax.ShapeDtypeStruct((b, n), jnp.float32)
    f = pl.kernel(
        _sc_body,
        out_type=(out, out),
        mesh=mesh,
        scratch_types=[
            pltpu.VMEM((n,), jnp.float32),
            pltpu.VMEM((n,), jnp.float32),
            pltpu.VMEM((n,), jnp.float32),
            pltpu.VMEM((n // _U,), jnp.float32),
            pltpu.SMEM((n // (_U * _L),), jnp.int32),
            pltpu.SemaphoreType.DMA,
            pltpu.SemaphoreType.DMA,
        ],
        compiler_params=pltpu.CompilerParams(needs_layout_passes=False),
    )
    return f(x)


# speedup vs baseline: 1.5626x; 1.0212x over previous
import jax
import jax.numpy as jnp
from jax import lax
from jax.experimental import pallas as pl
from jax.experimental.pallas import tpu as pltpu
from jax.experimental.pallas import tpu_sc as plsc

_Z = 1.0
_NEWTON_ITERS = 16
_L = 16
_U = 8
_ROWS_PER_TEC = 4


def _sc_body(x_hbm, wp_hbm, wc_hbm, xbuf, wpbuf, wcbuf, glist, sem_wp, sem_wc):
    n = x_hbm.shape[-1]
    nchunks = n // _L
    ngroups = nchunks // _U
    wid = lax.axis_index("s") * 2 + lax.axis_index("c")

    pending = None
    for r in range(_ROWS_PER_TEC):
        row = wid * _ROWS_PER_TEC + r
        pltpu.sync_copy(x_hbm.at[row], xbuf)

        neg_inf = jnp.full((_L,), -jnp.inf, jnp.float32)

        def pa(g, ms):
            base = g * (_U * _L)
            return tuple(
                jnp.maximum(ms[u], xbuf[pl.ds(base + u * _L, _L)])
                for u in range(_U)
            )

        ms = lax.fori_loop(0, ngroups, pa, (neg_inf,) * _U)
        m = ms[0]
        for u in range(1, _U):
            m = jnp.maximum(m, ms[u])
        tau0 = jnp.full((_L,), jnp.max(m) - _Z, jnp.float32)

        if pending is not None:
            pending[1].wait()

        def pb(g, gcnt):
            base = g * (_U * _L)
            anym = xbuf[pl.ds(base, _L)] > tau0
            for u in range(1, _U):
                anym = anym | (xbuf[pl.ds(base + u * _L, _L)] > tau0)
            hit = jnp.any(anym)

            @pl.when(hit)
            def _():
                glist[gcnt] = g

            return gcnt + jnp.where(hit, 1, 0)

        ngr = lax.fori_loop(0, ngroups, pb, jnp.int32(0))

        zero_f = jnp.zeros((_L,), jnp.float32)

        def nstep(tau):
            def inner(i, acc):
                svs = list(acc[:_U])
                kvs = list(acc[_U:])
                base = glist[i] * (_U * _L)
                for u in range(_U):
                    v = xbuf[pl.ds(base + u * _L, _L)]
                    act = v > tau
                    svs[u] = svs[u] + jnp.where(act, v, 0.0)
                    kvs[u] = kvs[u] + jnp.where(act, 1.0, 0.0)
                return tuple(svs) + tuple(kvs)

            acc = lax.fori_loop(0, ngr, inner, (zero_f,) * (2 * _U))
            sv = acc[0]
            kv = acc[_U]
            for u in range(1, _U):
                sv = sv + acc[u]
                kv = kv + acc[_U + u]
            s = jnp.full((_L,), jnp.sum(sv), jnp.float32)
            k = jnp.full((_L,), jnp.sum(kv), jnp.float32)
            return (s - _Z) / k

        def ncond(st):
            i, tau, taup = st
            return (i < _NEWTON_ITERS) & jnp.any(tau != taup)

        def nbody(st):
            i, tau, _ = st
            return (i + 1, nstep(tau), tau)

        _, tau, _ = lax.while_loop(
            ncond, nbody, (jnp.int32(0), nstep(tau0), tau0)
        )

        if pending is not None:
            pending[0].wait()

        def pc(g, _):
            base = g * (_U * _L)
            for u in range(_U):
                sl = pl.ds(base + u * _L, _L)
                v = xbuf[sl]
                wpbuf[sl] = jnp.maximum(v - tau, 0.0)
                wcbuf[sl] = jnp.minimum(v, tau)
            return 0

        lax.fori_loop(0, ngroups, pc, 0)
        cp_wp = pltpu.async_copy(wpbuf, wp_hbm.at[row], sem_wp)
        cp_wc = pltpu.async_copy(wcbuf, wc_hbm.at[row], sem_wc)
        pending = (cp_wp, cp_wc)

    pending[0].wait()
    pending[1].wait()


def kernel(x):
    b, n = x.shape
    mesh = plsc.VectorSubcoreMesh(core_axis_name="c", subcore_axis_name="s")
    out = jax.ShapeDtypeStruct((b, n), jnp.float32)
    f = pl.kernel(
        _sc_body,
        out_type=(out, out),
        mesh=mesh,
        scratch_types=[
            pltpu.VMEM((n,), jnp.float32),
            pltpu.VMEM((n,), jnp.float32),
            pltpu.VMEM((n,), jnp.float32),
            pltpu.SMEM((n // (_U * _L),), jnp.int32),
            pltpu.SemaphoreType.DMA,
            pltpu.SemaphoreType.DMA,
        ],
        compiler_params=pltpu.CompilerParams(needs_layout_passes=False),
    )
    return f(x)
